# baseline (speedup 1.0000x reference)
import functools

import jax
import jax.numpy as jnp
from jax import lax
from jax.experimental import pallas as pl
from jax.experimental.pallas import tpu as pltpu
from jax.experimental.pallas import tpu_sc as plsc

N = 10000
E = 320000
H = 128
HH = 64
NB = 23
NGRAPH = 64
FEAT = 27

NC = 2
NS = 16
NW = NC * NS
EPW = E // NW
NSPLIT = 2
EP = E // NSPLIT
EPP = EP // NW
NPAD = 10112
RPS = NPAD // NS
CH4 = 40
NCH4 = EPP // CH4
SEG4 = 25
NSEG4 = NCH4 // SEG4
NSP6 = 5
EP6 = E // NSP6
NCH6 = EP6 // NW // CH4
BN = 1000
BE = 4000
BE7 = 2000

_f32 = jnp.float32


def _seg_body(batch_ref, pos_ref, out_ref):
    i = pl.program_id(0)
    oh = (batch_ref[...].reshape(BN, 1)
          == lax.broadcasted_iota(jnp.int32, (BN, NGRAPH), 1))
    contrib = lax.dot_general(oh.astype(_f32), pos_ref[...],
                              (((0,), (0,)), ((), ())),
                              preferred_element_type=_f32)

    @pl.when(i == 0)
    def _():
        out_ref[...] = jnp.zeros_like(out_ref)

    out_ref[...] += contrib


def _node_body(batch_ref, pos_ref, seg_ref, atom_ref, rf_ref, pf_ref,
               atab_ref, wfeat_ref, wmsg_ref, posc_ref, z_ref, zmsg_ref):
    seg = seg_ref[...]
    mean = seg / jnp.maximum(seg[:, 3:4], 1.0)
    oh = (batch_ref[...].reshape(BN, 1)
          == lax.broadcasted_iota(jnp.int32, (BN, NGRAPH), 1))
    posc_ref[...] = pos_ref[...] - jnp.dot(oh.astype(_f32), mean,
                                           preferred_element_type=_f32)
    oha = (atom_ref[...].reshape(BN, 1)
           == lax.broadcasted_iota(jnp.int32, (BN, 128), 1))
    aemb = jnp.dot(oha.astype(_f32), atab_ref[...], preferred_element_type=_f32)
    er = jnp.dot(rf_ref[...], wfeat_ref[...], preferred_element_type=_f32)
    ep = jnp.dot(pf_ref[...], wfeat_ref[...], preferred_element_type=_f32)
    z = jnp.concatenate([aemb + er, ep - er], axis=-1)
    z_ref[...] = z
    zmsg_ref[...] = jnp.dot(z, wmsg_ref[...], preferred_element_type=_f32)


def _eattr_body(lsq_ref, bt_ref, wlen_ref, blen_ref, br_ref, bp_ref,
                w1a_ref, w1b_ref, b1_ref, w2_ref, b2_ref, out_ref):
    l = jnp.sqrt(lsq_ref[...].reshape(BE, 1) + 1e-12)
    lemb = jnp.tanh(l * wlen_ref[...] + blen_ref[...])
    ohb = (bt_ref[...].reshape(BE, 1)
           == lax.broadcasted_iota(jnp.int32, (BE, 32), 1))
    ohb = ohb.astype(_f32)
    br = jnp.dot(ohb, br_ref[...], preferred_element_type=_f32)
    bp = jnp.dot(ohb, bp_ref[...], preferred_element_type=_f32)
    attr_r = lemb * br
    attr_p = lemb * bp
    cat1 = jnp.dot(attr_r, w1a_ref[...], preferred_element_type=_f32)
    cat1 += jnp.dot(attr_p, w1b_ref[...], preferred_element_type=_f32)
    cat1 = jnp.maximum(cat1 + b1_ref[...], 0.0)
    out_ref[...] = jnp.dot(cat1, w2_ref[...], preferred_element_type=_f32) + b2_ref[...]


def _h_body(a0_ref, a1_ref, a2_ref, a3_ref, z_ref, wupd_ref,
            w1a_ref, w1b_ref, bg1_ref, hA_ref, hB_ref):
    agg = ((a0_ref[...] + a1_ref[...])
           + (a2_ref[...] + a3_ref[...])).reshape(BN, H)
    h = z_ref[...] + jnp.tanh(jnp.dot(agg, wupd_ref[...],
                                      preferred_element_type=_f32))
    hA_ref[...] = jnp.dot(h, w1a_ref[...], preferred_element_type=_f32) + bg1_ref[...]
    hB_ref[...] = jnp.dot(h, w1b_ref[...], preferred_element_type=_f32)


def _out_body(g_ref, wg2_ref, bg2_ref, wg3_ref, bg3_ref, o_ref):
    g1 = jnp.maximum(g_ref[...], 0.0)
    g2 = jnp.maximum(jnp.dot(g1, wg2_ref[...], preferred_element_type=_f32)
                     + bg2_ref[...], 0.0)

    o_ref[...] = (jnp.sum(g2 * wg3_ref[...], axis=1)
                  + bg3_ref[0, 0]).reshape(1, 1, BE7)


_MESH = plsc.VectorSubcoreMesh(core_axis_name="c", subcore_axis_name="s")
_SC_PARAMS = pltpu.CompilerParams(needs_layout_passes=False)


@functools.partial(
    pl.kernel,
    out_type=jax.ShapeDtypeStruct((E,), _f32),
    mesh=_MESH,
    compiler_params=_SC_PARAMS,
    scratch_types=[
        pltpu.VMEM((3 * N,), _f32),
        pltpu.VMEM((EPW,), jnp.int32),
        pltpu.VMEM((EPW,), jnp.int32),
        pltpu.VMEM((EPW,), _f32),
    ],
)
def _k2_lensq(posf_hbm, src_hbm, dst_hbm, lensq_hbm,
              posf_v, sidx_v, didx_v, out_v):
    c = lax.axis_index("c")
    s = lax.axis_index("s")
    base = (s * NC + c) * EPW
    pltpu.sync_copy(posf_hbm, posf_v)
    pltpu.sync_copy(src_hbm.at[pl.ds(base, EPW)], sidx_v)
    pltpu.sync_copy(dst_hbm.at[pl.ds(base, EPW)], didx_v)

    def body(i, carry):
        si = sidx_v[pl.ds(i * 16, 16)]
        di = didx_v[pl.ds(i * 16, 16)]
        xs = plsc.load_gather(posf_v, [si])
        ys = plsc.load_gather(posf_v, [si + N])
        zs = plsc.load_gather(posf_v, [si + 2 * N])
        xd = plsc.load_gather(posf_v, [di])
        yd = plsc.load_gather(posf_v, [di + N])
        zd = plsc.load_gather(posf_v, [di + 2 * N])
        dx = xd - xs
        dy = yd - ys
        dz = zd - zs
        out_v[pl.ds(i * 16, 16)] = dx * dx + dy * dy + dz * dz
        return carry

    lax.fori_loop(0, EPW // 16, body, 0)
    pltpu.sync_copy(out_v, lensq_hbm.at[pl.ds(base, EPW)])


@functools.partial(
    pl.kernel,
    out_type=jax.ShapeDtypeStruct((NC, NPAD, H), _f32),
    mesh=_MESH,
    compiler_params=_SC_PARAMS,
    scratch_types=[
        pltpu.VMEM((SEG4 * CH4,), jnp.int32),
        pltpu.VMEM((SEG4, CH4), jnp.int32),
        pltpu.VMEM((CH4, H), _f32),
        pltpu.VMEM((CH4, H), _f32),
        pltpu.VMEM((CH4, H), _f32),
        pltpu.VMEM((CH4, H), _f32),
        pltpu.VMEM_SHARED((NPAD, H), _f32),
        pltpu.SemaphoreType.DMA,
        pltpu.SemaphoreType.DMA,
        pltpu.SemaphoreType.DMA,
        pltpu.SemaphoreType.DMA,
        pltpu.SemaphoreType.DMA,
        pltpu.SemaphoreType.DMA,
    ],
)
def _k4_agg(zmsg_hbm, eattr_hbm, src3_hbm, dst3_hbm, agg_hbm,
            sidxseg, didxseg, zbuf0, zbuf1, ebuf0, ebuf1, agg_sh,
            semz0, semz1, seme0, seme1, semc0, semc1):
    c = lax.axis_index("c")
    s = lax.axis_index("s")
    wid = s * NC + c
    zbufs = (zbuf0, zbuf1)
    ebufs = (ebuf0, ebuf1)
    semzs = (semz0, semz1)
    semes = (seme0, seme1)
    semcs = (semc0, semc1)

    def zrow(e, carry):
        for q in range(8):
            zbuf0[e, pl.ds(q * 16, 16)] = jnp.zeros((16,), _f32)
        return carry

    lax.fori_loop(0, CH4, zrow, 0)
    for k in range(RPS // CH4):
        pltpu.sync_copy(zbuf0, agg_sh.at[pl.ds(s * RPS + k * CH4, CH4)])
    pltpu.sync_copy(zbuf0.at[pl.ds(0, RPS % CH4)],
                    agg_sh.at[pl.ds(s * RPS + (RPS // CH4) * CH4, RPS % CH4)])
    plsc.subcore_barrier()

    def start(g, j, p):
        pltpu.async_copy(zmsg_hbm.at[sidxseg.at[pl.ds(j * CH4, CH4)]],
                         zbufs[p], semzs[p])
        eb = (wid * NCH4 + g * SEG4 + j) * CH4
        pltpu.async_copy(eattr_hbm.at[pl.ds(eb, CH4)], ebufs[p], semes[p])

    def finish(g, j, p):
        pltpu.make_async_copy(zmsg_hbm.at[sidxseg.at[pl.ds(j * CH4, CH4)]],
                              zbufs[p], semzs[p]).wait()
        eb = (wid * NCH4 + g * SEG4 + j) * CH4
        pltpu.make_async_copy(eattr_hbm.at[pl.ds(eb, CH4)], ebufs[p],
                              semes[p]).wait()

        def mrow(e, cc):
            for q in range(8):
                sl = pl.ds(q * 16, 16)
                zbufs[p][e, sl] = zbufs[p][e, sl] * ebufs[p][e, sl]
            return cc

        lax.fori_loop(0, CH4, mrow, 0)
        pltpu.async_copy(zbufs[p], agg_sh.at[didxseg.at[j]], semcs[p],
                         add=True)

    def wait_scat(j, p):
        pltpu.make_async_copy(zbufs[p], agg_sh.at[didxseg.at[j]],
                              semcs[p]).wait()

    def seg(g, carry):
        pltpu.sync_copy(
            src3_hbm.at[pl.ds(wid * EPP + g * SEG4 * CH4, SEG4 * CH4)],
            sidxseg)
        pltpu.sync_copy(dst3_hbm.at[wid, g], didxseg)
        start(g, 0, 0)
        start(g, 1, 1)
        finish(g, 0, 0)
        wait_scat(0, 0)
        start(g, 2, 0)
        finish(g, 1, 1)

        def body(i, cc):
            a = 2 * i
            wait_scat(a - 1, 1)
            start(g, a + 1, 1)
            finish(g, a, 0)
            wait_scat(a, 0)
            start(g, a + 2, 0)
            finish(g, a + 1, 1)
            return cc

        lax.fori_loop(1, (SEG4 - 1) // 2, body, 0)
        wait_scat(SEG4 - 2, 1)
        finish(g, SEG4 - 1, 0)
        wait_scat(SEG4 - 1, 0)
        return carry

    lax.fori_loop(0, NSEG4, seg, 0)
    plsc.subcore_barrier()
    for k in range(RPS // CH4):
        r0 = s * RPS + k * CH4
        pltpu.sync_copy(agg_sh.at[pl.ds(r0, CH4)], zbuf0)
        pltpu.sync_copy(zbuf0, agg_hbm.at[c, pl.ds(r0, CH4)])
    r0 = s * RPS + (RPS // CH4) * CH4
    pltpu.sync_copy(agg_sh.at[pl.ds(r0, RPS % CH4)], zbuf0.at[pl.ds(0, RPS % CH4)])
    pltpu.sync_copy(zbuf0.at[pl.ds(0, RPS % CH4)], agg_hbm.at[c, pl.ds(r0, RPS % CH4)])


@functools.partial(
    pl.kernel,
    out_type=jax.ShapeDtypeStruct((EP6, H), _f32),
    mesh=_MESH,
    compiler_params=_SC_PARAMS,
    scratch_types=[
        pltpu.VMEM((NCH6 * CH4,), jnp.int32),
        pltpu.VMEM((NCH6 * CH4,), jnp.int32),
        pltpu.VMEM((CH4, H), _f32),
        pltpu.VMEM((CH4, H), _f32),
        pltpu.VMEM((CH4, H), _f32),
        pltpu.VMEM((CH4, H), _f32),
        pltpu.SemaphoreType.DMA,
        pltpu.SemaphoreType.DMA,
        pltpu.SemaphoreType.DMA,
        pltpu.SemaphoreType.DMA,
    ],
)
def _k6_pair(hA_hbm, hB_hbm, src3_hbm, dst3_hbm, gsum_hbm,
             sidx2, didx2, abuf0, abuf1, bbuf0, bbuf1,
             sema0, sema1, semb0, semb1):
    c = lax.axis_index("c")
    s = lax.axis_index("s")
    wid = s * NC + c
    abufs = (abuf0, abuf1)
    bbufs = (bbuf0, bbuf1)
    semas = (sema0, sema1)
    sembs = (semb0, semb1)

    epw6 = NCH6 * CH4
    pltpu.sync_copy(src3_hbm.at[pl.ds(wid * epw6, epw6)], sidx2)
    pltpu.sync_copy(dst3_hbm.at[pl.ds(wid * epw6, epw6)], didx2)

    def start(j, p):
        pltpu.async_copy(hA_hbm.at[sidx2.at[pl.ds(j * CH4, CH4)]],
                         abufs[p], semas[p])
        pltpu.async_copy(hB_hbm.at[didx2.at[pl.ds(j * CH4, CH4)]],
                         bbufs[p], sembs[p])

    def finish(j, p):
        pltpu.make_async_copy(hA_hbm.at[sidx2.at[pl.ds(j * CH4, CH4)]],
                              abufs[p], semas[p]).wait()
        pltpu.make_async_copy(hB_hbm.at[didx2.at[pl.ds(j * CH4, CH4)]],
                              bbufs[p], sembs[p]).wait()

        def arow(e, cc):
            for q in range(8):
                sl = pl.ds(q * 16, 16)
                abufs[p][e, sl] = abufs[p][e, sl] + bbufs[p][e, sl]
            return cc

        lax.fori_loop(0, CH4, arow, 0)
        pltpu.sync_copy(abufs[p],
                        gsum_hbm.at[pl.ds((wid * NCH6 + j) * CH4, CH4)])

    start(0, 0)

    def body(i, carry):
        a = 2 * i
        start(a + 1, 1)
        finish(a, 0)
        start(a + 2, 0)
        finish(a + 1, 1)
        return carry

    lax.fori_loop(0, NCH6 // 2 - 1, body, 0)
    start(NCH6 - 1, 1)
    finish(NCH6 - 2, 0)
    finish(NCH6 - 1, 1)


def kernel(atom_type, r_feat, p_feat, pos, bond_index, bond_type, batch,
           atom_table, W_feat, bond_emb_r, bond_emb_p, W_len, b_len,
           W_cat1, b_cat1, W_cat2, b_cat2, W_msg, W_upd,
           Wg1, bg1, Wg2, bg2, Wg3, bg3):
    batch2 = batch.astype(jnp.int32).reshape(N // BN, 1, BN)
    atom2 = atom_type.astype(jnp.int32).reshape(N // BN, 1, BN)
    pos16 = jnp.concatenate(
        [pos.astype(_f32), jnp.ones((N, 1), _f32), jnp.zeros((N, 12), _f32)],
        axis=1)
    atab = jnp.concatenate([atom_table, jnp.zeros((28, HH), _f32)], axis=0)
    br32 = jnp.concatenate([bond_emb_r, jnp.zeros((32 - NB, H), _f32)], axis=0)
    bp32 = jnp.concatenate([bond_emb_p, jnp.zeros((32 - NB, H), _f32)], axis=0)
    w1a, w1b = W_cat1[:H], W_cat1[H:]
    wg1a, wg1b = Wg1[:H], Wg1[H:]
    src = bond_index[0].astype(jnp.int32)
    dst = bond_index[1].astype(jnp.int32)
    srcp = [src[p * EP:(p + 1) * EP] for p in range(NSPLIT)]
    dstp = [dst[p * EP:(p + 1) * EP] for p in range(NSPLIT)]
    bond1 = bond_type.astype(jnp.int32)
    blen = b_len.reshape(1, H)
    b1 = b_cat1.reshape(1, H)
    b2 = b_cat2.reshape(1, H)
    bg1r = bg1.reshape(1, H)
    bg2r = bg2.reshape(1, HH)
    wg3r = Wg3.reshape(1, HH)
    bg3r = bg3.reshape(1, 1)

    seg = pl.pallas_call(
        _seg_body,
        grid=(N // BN,),
        in_specs=[pl.BlockSpec((1, 1, BN), lambda i: (i, 0, 0)),
                  pl.BlockSpec((BN, 16), lambda i: (i, 0))],
        out_specs=pl.BlockSpec((NGRAPH, 16), lambda i: (0, 0)),
        out_shape=jax.ShapeDtypeStruct((NGRAPH, 16), _f32),
    )(batch2, pos16)

    posc, z, zmsg = pl.pallas_call(
        _node_body,
        grid=(N // BN,),
        in_specs=[pl.BlockSpec((1, 1, BN), lambda i: (i, 0, 0)),
                  pl.BlockSpec((BN, 16), lambda i: (i, 0)),
                  pl.BlockSpec((NGRAPH, 16), lambda i: (0, 0)),
                  pl.BlockSpec((1, 1, BN), lambda i: (i, 0, 0)),
                  pl.BlockSpec((BN, FEAT), lambda i: (i, 0)),
                  pl.BlockSpec((BN, FEAT), lambda i: (i, 0)),
                  pl.BlockSpec((128, HH), lambda i: (0, 0)),
                  pl.BlockSpec((FEAT, HH), lambda i: (0, 0)),
                  pl.BlockSpec((H, H), lambda i: (0, 0))],
        out_specs=[pl.BlockSpec((BN, 16), lambda i: (i, 0)),
                   pl.BlockSpec((BN, H), lambda i: (i, 0)),
                   pl.BlockSpec((BN, H), lambda i: (i, 0))],
        out_shape=[jax.ShapeDtypeStruct((N, 16), _f32),
                   jax.ShapeDtypeStruct((N, H), _f32),
                   jax.ShapeDtypeStruct((N, H), _f32)],
    )(batch2, pos16, seg, atom2, r_feat, p_feat, atab, W_feat, W_msg)

    posf = posc[:, :3].T.reshape(3 * N)
    lensq = _k2_lensq(posf, src, dst)

    def run_k3(lsq_p, bond_p):
        return pl.pallas_call(
            _eattr_body,
            grid=(EP // BE,),
            in_specs=[pl.BlockSpec((1, 1, BE), lambda i: (i, 0, 0)),
                      pl.BlockSpec((1, 1, BE), lambda i: (i, 0, 0)),
                      pl.BlockSpec((1, H), lambda i: (0, 0)),
                      pl.BlockSpec((1, H), lambda i: (0, 0)),
                      pl.BlockSpec((32, H), lambda i: (0, 0)),
                      pl.BlockSpec((32, H), lambda i: (0, 0)),
                      pl.BlockSpec((H, H), lambda i: (0, 0)),
                      pl.BlockSpec((H, H), lambda i: (0, 0)),
                      pl.BlockSpec((1, H), lambda i: (0, 0)),
                      pl.BlockSpec((H, H), lambda i: (0, 0)),
                      pl.BlockSpec((1, H), lambda i: (0, 0))],
            out_specs=pl.BlockSpec((BE, H), lambda i: (i, 0)),
            out_shape=jax.ShapeDtypeStruct((EP, H), _f32),
        )(lsq_p.reshape(EP // BE, 1, BE), bond_p.reshape(EP // BE, 1, BE),
          W_len, blen, br32, bp32, w1a, w1b, b1, W_cat2, b2)

    aggs = []
    for p in range(NSPLIT):
        eattr_p = run_k3(lensq[p * EP:(p + 1) * EP],
                         bond1[p * EP:(p + 1) * EP])
        aggs.append(_k4_agg(zmsg, eattr_p, srcp[p],
                            dstp[p].reshape(NW, NSEG4, SEG4, CH4)))

    hA, hB = pl.pallas_call(
        _h_body,
        grid=(N // BN,),
        in_specs=[pl.BlockSpec((1, BN, H), lambda i: (0, i, 0)),
                  pl.BlockSpec((1, BN, H), lambda i: (1, i, 0)),
                  pl.BlockSpec((1, BN, H), lambda i: (0, i, 0)),
                  pl.BlockSpec((1, BN, H), lambda i: (1, i, 0)),
                  pl.BlockSpec((BN, H), lambda i: (i, 0)),
                  pl.BlockSpec((H, H), lambda i: (0, 0)),
                  pl.BlockSpec((H, H), lambda i: (0, 0)),
                  pl.BlockSpec((H, H), lambda i: (0, 0)),
                  pl.BlockSpec((1, H), lambda i: (0, 0))],
        out_specs=[pl.BlockSpec((BN, H), lambda i: (i, 0)),
                   pl.BlockSpec((BN, H), lambda i: (i, 0))],
        out_shape=[jax.ShapeDtypeStruct((N, H), _f32),
                   jax.ShapeDtypeStruct((N, H), _f32)],
    )(aggs[0], aggs[0], aggs[1], aggs[1],
      z, W_upd, wg1a, wg1b, bg1r)

    def run_k7(gsum_p):
        return pl.pallas_call(
            _out_body,
            grid=(EP6 // BE7,),
            in_specs=[pl.BlockSpec((BE7, H), lambda i: (i, 0)),
                      pl.BlockSpec((H, HH), lambda i: (0, 0)),
                      pl.BlockSpec((1, HH), lambda i: (0, 0)),
                      pl.BlockSpec((1, HH), lambda i: (0, 0)),
                      pl.BlockSpec((1, 1), lambda i: (0, 0))],
            out_specs=pl.BlockSpec((1, 1, BE7), lambda i: (i, 0, 0)),
            out_shape=jax.ShapeDtypeStruct((EP6 // BE7, 1, BE7), _f32),
        )(gsum_p, Wg2, bg2r, wg3r, bg3r)

    gparts = []
    for p in range(NSP6):
        gsum_p = _k6_pair(hA, hB,
                          src[p * EP6:(p + 1) * EP6],
                          dst[p * EP6:(p + 1) * EP6])
        gparts.append(run_k7(gsum_p).reshape(EP6))

    return jnp.concatenate(gparts).reshape(E, 1)


# speedup vs baseline: 1.0020x; 1.0020x over previous
import functools

import jax
import jax.numpy as jnp
from jax import lax
from jax.experimental import pallas as pl
from jax.experimental.pallas import tpu as pltpu
from jax.experimental.pallas import tpu_sc as plsc

N = 10000
E = 320000
H = 128
HH = 64
NB = 23
NGRAPH = 64
FEAT = 27

NC = 2
NS = 16
NW = NC * NS
EPW = E // NW
NSPLIT = 2
EP = E // NSPLIT
EPP = EP // NW
NPAD = 10112
RPS = NPAD // NS
CH4 = 40
NCH4 = EPP // CH4
SEG4 = 25
NSEG4 = NCH4 // SEG4
NSP6 = 5
EP6 = E // NSP6
NCH6 = EP6 // NW // CH4
BN = 1000
BE = 4000
BE7 = 2000

_f32 = jnp.float32


def _seg_body(batch_ref, pos_ref, out_ref):
    i = pl.program_id(0)
    oh = (batch_ref[...].reshape(BN, 1)
          == lax.broadcasted_iota(jnp.int32, (BN, NGRAPH), 1))
    contrib = lax.dot_general(oh.astype(_f32), pos_ref[...],
                              (((0,), (0,)), ((), ())),
                              preferred_element_type=_f32)

    @pl.when(i == 0)
    def _():
        out_ref[...] = jnp.zeros_like(out_ref)

    out_ref[...] += contrib


def _node_body(batch_ref, pos_ref, seg_ref, atom_ref, rf_ref, pf_ref,
               atab_ref, wfeat_ref, wmsg_ref, posc_ref, z_ref, zmsg_ref):
    seg = seg_ref[...]
    mean = seg / jnp.maximum(seg[:, 3:4], 1.0)
    oh = (batch_ref[...].reshape(BN, 1)
          == lax.broadcasted_iota(jnp.int32, (BN, NGRAPH), 1))
    posc_ref[...] = pos_ref[...] - jnp.dot(oh.astype(_f32), mean,
                                           preferred_element_type=_f32)
    oha = (atom_ref[...].reshape(BN, 1)
           == lax.broadcasted_iota(jnp.int32, (BN, 128), 1))
    aemb = jnp.dot(oha.astype(_f32), atab_ref[...], preferred_element_type=_f32)
    er = jnp.dot(rf_ref[...], wfeat_ref[...], preferred_element_type=_f32)
    ep = jnp.dot(pf_ref[...], wfeat_ref[...], preferred_element_type=_f32)
    z = jnp.concatenate([aemb + er, ep - er], axis=-1)
    z_ref[...] = z
    zmsg_ref[...] = jnp.dot(z, wmsg_ref[...], preferred_element_type=_f32)


def _eattr_body(lsq_ref, bt_ref, wlen_ref, blen_ref, br_ref, bp_ref,
                w1a_ref, w1b_ref, b1_ref, w2_ref, b2_ref, out_ref):
    l = jnp.sqrt(lsq_ref[...].reshape(BE, 1) + 1e-12)
    lemb = jnp.tanh(l * wlen_ref[...] + blen_ref[...])
    ohb = (bt_ref[...].reshape(BE, 1)
           == lax.broadcasted_iota(jnp.int32, (BE, 32), 1))
    ohb = ohb.astype(_f32)
    br = jnp.dot(ohb, br_ref[...], preferred_element_type=_f32)
    bp = jnp.dot(ohb, bp_ref[...], preferred_element_type=_f32)
    attr_r = lemb * br
    attr_p = lemb * bp
    cat1 = jnp.dot(attr_r, w1a_ref[...], preferred_element_type=_f32)
    cat1 += jnp.dot(attr_p, w1b_ref[...], preferred_element_type=_f32)
    cat1 = jnp.maximum(cat1 + b1_ref[...], 0.0)
    out_ref[...] = jnp.dot(cat1, w2_ref[...], preferred_element_type=_f32) + b2_ref[...]


def _h_body(a0_ref, a1_ref, a2_ref, a3_ref, z_ref, wupd_ref,
            w1a_ref, w1b_ref, bg1_ref, hA_ref, hB_ref):
    agg = ((a0_ref[...] + a1_ref[...])
           + (a2_ref[...] + a3_ref[...])).reshape(BN, H)
    h = z_ref[...] + jnp.tanh(jnp.dot(agg, wupd_ref[...],
                                      preferred_element_type=_f32))
    hA_ref[...] = jnp.dot(h, w1a_ref[...], preferred_element_type=_f32) + bg1_ref[...]
    hB_ref[...] = jnp.dot(h, w1b_ref[...], preferred_element_type=_f32)


def _out_body(g_ref, wg2_ref, bg2_ref, wg3_ref, bg3_ref, o_ref):
    g1 = jnp.maximum(g_ref[...], 0.0)
    g2 = jnp.maximum(jnp.dot(g1, wg2_ref[...], preferred_element_type=_f32)
                     + bg2_ref[...], 0.0)

    o_ref[...] = (jnp.sum(g2 * wg3_ref[...], axis=1)
                  + bg3_ref[0, 0]).reshape(1, 1, BE7)


_MESH = plsc.VectorSubcoreMesh(core_axis_name="c", subcore_axis_name="s")
_SC_PARAMS = pltpu.CompilerParams(needs_layout_passes=False)


@functools.partial(
    pl.kernel,
    out_type=jax.ShapeDtypeStruct((E,), _f32),
    mesh=_MESH,
    compiler_params=_SC_PARAMS,
    scratch_types=[
        pltpu.VMEM((3 * N,), _f32),
        pltpu.VMEM((EPW,), jnp.int32),
        pltpu.VMEM((EPW,), jnp.int32),
        pltpu.VMEM((EPW,), _f32),
    ],
)
def _k2_lensq(posf_hbm, src_hbm, dst_hbm, lensq_hbm,
              posf_v, sidx_v, didx_v, out_v):
    c = lax.axis_index("c")
    s = lax.axis_index("s")
    base = (s * NC + c) * EPW
    pltpu.sync_copy(posf_hbm, posf_v)
    pltpu.sync_copy(src_hbm.at[pl.ds(base, EPW)], sidx_v)
    pltpu.sync_copy(dst_hbm.at[pl.ds(base, EPW)], didx_v)

    def body(i, carry):
        si = sidx_v[pl.ds(i * 16, 16)]
        di = didx_v[pl.ds(i * 16, 16)]
        xs = plsc.load_gather(posf_v, [si])
        ys = plsc.load_gather(posf_v, [si + N])
        zs = plsc.load_gather(posf_v, [si + 2 * N])
        xd = plsc.load_gather(posf_v, [di])
        yd = plsc.load_gather(posf_v, [di + N])
        zd = plsc.load_gather(posf_v, [di + 2 * N])
        dx = xd - xs
        dy = yd - ys
        dz = zd - zs
        out_v[pl.ds(i * 16, 16)] = dx * dx + dy * dy + dz * dz
        return carry

    lax.fori_loop(0, EPW // 16, body, 0)
    pltpu.sync_copy(out_v, lensq_hbm.at[pl.ds(base, EPW)])


@functools.partial(
    pl.kernel,
    out_type=jax.ShapeDtypeStruct((NC, NPAD, H), _f32),
    mesh=_MESH,
    compiler_params=_SC_PARAMS,
    scratch_types=[
        pltpu.VMEM((SEG4 * CH4,), jnp.int32),
        pltpu.VMEM((SEG4, CH4), jnp.int32),
        pltpu.VMEM((CH4, H), _f32),
        pltpu.VMEM((CH4, H), _f32),
        pltpu.VMEM((CH4, H), _f32),
        pltpu.VMEM((CH4, H), _f32),
        pltpu.VMEM_SHARED((NPAD, H), _f32),
        pltpu.SemaphoreType.DMA,
        pltpu.SemaphoreType.DMA,
        pltpu.SemaphoreType.DMA,
        pltpu.SemaphoreType.DMA,
    ],
)
def _k4_agg(zmsg_hbm, eattr_hbm, src3_hbm, dst3_hbm, agg_hbm,
            sidxseg, didxseg, zbuf0, zbuf1, ebuf0, ebuf1, agg_sh,
            semz0, semz1, seme0, seme1):
    c = lax.axis_index("c")
    s = lax.axis_index("s")
    wid = s * NC + c
    zbufs = (zbuf0, zbuf1)
    ebufs = (ebuf0, ebuf1)
    semzs = (semz0, semz1)
    semes = (seme0, seme1)

    def zrow(e, carry):
        for q in range(8):
            zbuf0[e, pl.ds(q * 16, 16)] = jnp.zeros((16,), _f32)
        return carry

    lax.fori_loop(0, CH4, zrow, 0)
    for k in range(RPS // CH4):
        pltpu.sync_copy(zbuf0, agg_sh.at[pl.ds(s * RPS + k * CH4, CH4)])
    pltpu.sync_copy(zbuf0.at[pl.ds(0, RPS % CH4)],
                    agg_sh.at[pl.ds(s * RPS + (RPS // CH4) * CH4, RPS % CH4)])
    plsc.subcore_barrier()

    def start(g, j, p):
        pltpu.async_copy(zmsg_hbm.at[sidxseg.at[pl.ds(j * CH4, CH4)]],
                         zbufs[p], semzs[p])
        eb = (wid * NCH4 + g * SEG4 + j) * CH4
        pltpu.async_copy(eattr_hbm.at[pl.ds(eb, CH4)], ebufs[p], semes[p])

    def finish(g, j, p):
        pltpu.make_async_copy(zmsg_hbm.at[sidxseg.at[pl.ds(j * CH4, CH4)]],
                              zbufs[p], semzs[p]).wait()
        eb = (wid * NCH4 + g * SEG4 + j) * CH4
        pltpu.make_async_copy(eattr_hbm.at[pl.ds(eb, CH4)], ebufs[p],
                              semes[p]).wait()

        def mrow(e, cc):
            for q in range(8):
                sl = pl.ds(q * 16, 16)
                zbufs[p][e, sl] = zbufs[p][e, sl] * ebufs[p][e, sl]
            return cc

        lax.fori_loop(0, CH4, mrow, 0)
        pltpu.sync_copy(zbufs[p], agg_sh.at[didxseg.at[j]], add=True)

    def seg(g, carry):
        pltpu.sync_copy(
            src3_hbm.at[pl.ds(wid * EPP + g * SEG4 * CH4, SEG4 * CH4)],
            sidxseg)
        pltpu.sync_copy(dst3_hbm.at[wid, g], didxseg)
        start(g, 0, 0)

        def body(i, cc):
            a = 2 * i
            start(g, a + 1, 1)
            finish(g, a, 0)
            start(g, a + 2, 0)
            finish(g, a + 1, 1)
            return cc

        lax.fori_loop(0, (SEG4 - 1) // 2, body, 0)
        finish(g, SEG4 - 1, 0)
        return carry

    lax.fori_loop(0, NSEG4, seg, 0)
    plsc.subcore_barrier()
    for k in range(RPS // CH4):
        r0 = s * RPS + k * CH4
        pltpu.sync_copy(agg_sh.at[pl.ds(r0, CH4)], zbuf0)
        pltpu.sync_copy(zbuf0, agg_hbm.at[c, pl.ds(r0, CH4)])
    r0 = s * RPS + (RPS // CH4) * CH4
    pltpu.sync_copy(agg_sh.at[pl.ds(r0, RPS % CH4)], zbuf0.at[pl.ds(0, RPS % CH4)])
    pltpu.sync_copy(zbuf0.at[pl.ds(0, RPS % CH4)], agg_hbm.at[c, pl.ds(r0, RPS % CH4)])


@functools.partial(
    pl.kernel,
    out_type=jax.ShapeDtypeStruct((EP6, H), _f32),
    mesh=_MESH,
    compiler_params=_SC_PARAMS,
    scratch_types=[
        pltpu.VMEM((NCH6 * CH4,), jnp.int32),
        pltpu.VMEM((NCH6 * CH4,), jnp.int32),
        pltpu.VMEM((CH4, H), _f32),
        pltpu.VMEM((CH4, H), _f32),
        pltpu.VMEM((CH4, H), _f32),
        pltpu.VMEM((CH4, H), _f32),
        pltpu.SemaphoreType.DMA,
        pltpu.SemaphoreType.DMA,
        pltpu.SemaphoreType.DMA,
        pltpu.SemaphoreType.DMA,
    ],
)
def _k6_pair(hA_hbm, hB_hbm, src3_hbm, dst3_hbm, gsum_hbm,
             sidx2, didx2, abuf0, abuf1, bbuf0, bbuf1,
             sema0, sema1, semb0, semb1):
    c = lax.axis_index("c")
    s = lax.axis_index("s")
    wid = s * NC + c
    abufs = (abuf0, abuf1)
    bbufs = (bbuf0, bbuf1)
    semas = (sema0, sema1)
    sembs = (semb0, semb1)

    epw6 = NCH6 * CH4
    pltpu.sync_copy(src3_hbm.at[pl.ds(wid * epw6, epw6)], sidx2)
    pltpu.sync_copy(dst3_hbm.at[pl.ds(wid * epw6, epw6)], didx2)

    def start(j, p):
        pltpu.async_copy(hA_hbm.at[sidx2.at[pl.ds(j * CH4, CH4)]],
                         abufs[p], semas[p])
        pltpu.async_copy(hB_hbm.at[didx2.at[pl.ds(j * CH4, CH4)]],
                         bbufs[p], sembs[p])

    def finish(j, p):
        pltpu.make_async_copy(hA_hbm.at[sidx2.at[pl.ds(j * CH4, CH4)]],
                              abufs[p], semas[p]).wait()
        pltpu.make_async_copy(hB_hbm.at[didx2.at[pl.ds(j * CH4, CH4)]],
                              bbufs[p], sembs[p]).wait()

        def arow(e, cc):
            for q in range(8):
                sl = pl.ds(q * 16, 16)
                abufs[p][e, sl] = abufs[p][e, sl] + bbufs[p][e, sl]
            return cc

        lax.fori_loop(0, CH4, arow, 0)
        pltpu.sync_copy(abufs[p],
                        gsum_hbm.at[pl.ds((wid * NCH6 + j) * CH4, CH4)])

    start(0, 0)

    def body(i, carry):
        a = 2 * i
        start(a + 1, 1)
        finish(a, 0)
        start(a + 2, 0)
        finish(a + 1, 1)
        return carry

    lax.fori_loop(0, NCH6 // 2 - 1, body, 0)
    start(NCH6 - 1, 1)
    finish(NCH6 - 2, 0)
    finish(NCH6 - 1, 1)


def kernel(atom_type, r_feat, p_feat, pos, bond_index, bond_type, batch,
           atom_table, W_feat, bond_emb_r, bond_emb_p, W_len, b_len,
           W_cat1, b_cat1, W_cat2, b_cat2, W_msg, W_upd,
           Wg1, bg1, Wg2, bg2, Wg3, bg3):
    batch2 = batch.astype(jnp.int32).reshape(N // BN, 1, BN)
    atom2 = atom_type.astype(jnp.int32).reshape(N // BN, 1, BN)
    pos16 = jnp.concatenate(
        [pos.astype(_f32), jnp.ones((N, 1), _f32), jnp.zeros((N, 12), _f32)],
        axis=1)
    atab = jnp.concatenate([atom_table, jnp.zeros((28, HH), _f32)], axis=0)
    br32 = jnp.concatenate([bond_emb_r, jnp.zeros((32 - NB, H), _f32)], axis=0)
    bp32 = jnp.concatenate([bond_emb_p, jnp.zeros((32 - NB, H), _f32)], axis=0)
    w1a, w1b = W_cat1[:H], W_cat1[H:]
    wg1a, wg1b = Wg1[:H], Wg1[H:]
    src = bond_index[0].astype(jnp.int32)
    dst = bond_index[1].astype(jnp.int32)
    srcp = [src[p * EP:(p + 1) * EP] for p in range(NSPLIT)]
    dstp = [dst[p * EP:(p + 1) * EP] for p in range(NSPLIT)]
    bond1 = bond_type.astype(jnp.int32)
    blen = b_len.reshape(1, H)
    b1 = b_cat1.reshape(1, H)
    b2 = b_cat2.reshape(1, H)
    bg1r = bg1.reshape(1, H)
    bg2r = bg2.reshape(1, HH)
    wg3r = Wg3.reshape(1, HH)
    bg3r = bg3.reshape(1, 1)

    seg = pl.pallas_call(
        _seg_body,
        grid=(N // BN,),
        in_specs=[pl.BlockSpec((1, 1, BN), lambda i: (i, 0, 0)),
                  pl.BlockSpec((BN, 16), lambda i: (i, 0))],
        out_specs=pl.BlockSpec((NGRAPH, 16), lambda i: (0, 0)),
        out_shape=jax.ShapeDtypeStruct((NGRAPH, 16), _f32),
    )(batch2, pos16)

    posc, z, zmsg = pl.pallas_call(
        _node_body,
        grid=(N // BN,),
        in_specs=[pl.BlockSpec((1, 1, BN), lambda i: (i, 0, 0)),
                  pl.BlockSpec((BN, 16), lambda i: (i, 0)),
                  pl.BlockSpec((NGRAPH, 16), lambda i: (0, 0)),
                  pl.BlockSpec((1, 1, BN), lambda i: (i, 0, 0)),
                  pl.BlockSpec((BN, FEAT), lambda i: (i, 0)),
                  pl.BlockSpec((BN, FEAT), lambda i: (i, 0)),
                  pl.BlockSpec((128, HH), lambda i: (0, 0)),
                  pl.BlockSpec((FEAT, HH), lambda i: (0, 0)),
                  pl.BlockSpec((H, H), lambda i: (0, 0))],
        out_specs=[pl.BlockSpec((BN, 16), lambda i: (i, 0)),
                   pl.BlockSpec((BN, H), lambda i: (i, 0)),
                   pl.BlockSpec((BN, H), lambda i: (i, 0))],
        out_shape=[jax.ShapeDtypeStruct((N, 16), _f32),
                   jax.ShapeDtypeStruct((N, H), _f32),
                   jax.ShapeDtypeStruct((N, H), _f32)],
    )(batch2, pos16, seg, atom2, r_feat, p_feat, atab, W_feat, W_msg)

    posf = posc[:, :3].T.reshape(3 * N)
    lensq = _k2_lensq(posf, src, dst)

    def run_k3(lsq_p, bond_p):
        return pl.pallas_call(
            _eattr_body,
            grid=(EP // BE,),
            in_specs=[pl.BlockSpec((1, 1, BE), lambda i: (i, 0, 0)),
                      pl.BlockSpec((1, 1, BE), lambda i: (i, 0, 0)),
                      pl.BlockSpec((1, H), lambda i: (0, 0)),
                      pl.BlockSpec((1, H), lambda i: (0, 0)),
                      pl.BlockSpec((32, H), lambda i: (0, 0)),
                      pl.BlockSpec((32, H), lambda i: (0, 0)),
                      pl.BlockSpec((H, H), lambda i: (0, 0)),
                      pl.BlockSpec((H, H), lambda i: (0, 0)),
                      pl.BlockSpec((1, H), lambda i: (0, 0)),
                      pl.BlockSpec((H, H), lambda i: (0, 0)),
                      pl.BlockSpec((1, H), lambda i: (0, 0))],
            out_specs=pl.BlockSpec((BE, H), lambda i: (i, 0)),
            out_shape=jax.ShapeDtypeStruct((EP, H), _f32),
        )(lsq_p.reshape(EP // BE, 1, BE), bond_p.reshape(EP // BE, 1, BE),
          W_len, blen, br32, bp32, w1a, w1b, b1, W_cat2, b2)

    aggs = []
    for p in range(NSPLIT):
        eattr_p = run_k3(lensq[p * EP:(p + 1) * EP],
                         bond1[p * EP:(p + 1) * EP])
        aggs.append(_k4_agg(zmsg, eattr_p, srcp[p],
                            dstp[p].reshape(NW, NSEG4, SEG4, CH4)))

    hA, hB = pl.pallas_call(
        _h_body,
        grid=(N // BN,),
        in_specs=[pl.BlockSpec((1, BN, H), lambda i: (0, i, 0)),
                  pl.BlockSpec((1, BN, H), lambda i: (1, i, 0)),
                  pl.BlockSpec((1, BN, H), lambda i: (0, i, 0)),
                  pl.BlockSpec((1, BN, H), lambda i: (1, i, 0)),
                  pl.BlockSpec((BN, H), lambda i: (i, 0)),
                  pl.BlockSpec((H, H), lambda i: (0, 0)),
                  pl.BlockSpec((H, H), lambda i: (0, 0)),
                  pl.BlockSpec((H, H), lambda i: (0, 0)),
                  pl.BlockSpec((1, H), lambda i: (0, 0))],
        out_specs=[pl.BlockSpec((BN, H), lambda i: (i, 0)),
                   pl.BlockSpec((BN, H), lambda i: (i, 0))],
        out_shape=[jax.ShapeDtypeStruct((N, H), _f32),
                   jax.ShapeDtypeStruct((N, H), _f32)],
    )(aggs[0], aggs[0], aggs[1], aggs[1],
      z, W_upd, wg1a, wg1b, bg1r)

    def run_k7(gsum_p):
        return pl.pallas_call(
            _out_body,
            grid=(EP6 // BE7,),
            in_specs=[pl.BlockSpec((BE7, H), lambda i: (i, 0)),
                      pl.BlockSpec((H, HH), lambda i: (0, 0)),
                      pl.BlockSpec((1, HH), lambda i: (0, 0)),
                      pl.BlockSpec((1, HH), lambda i: (0, 0)),
                      pl.BlockSpec((1, 1), lambda i: (0, 0))],
            out_specs=pl.BlockSpec((1, 1, BE7), lambda i: (i, 0, 0)),
            out_shape=jax.ShapeDtypeStruct((EP6 // BE7, 1, BE7), _f32),
        )(gsum_p, Wg2, bg2r, wg3r, bg3r)

    gparts = []
    for p in range(NSP6):
        gsum_p = _k6_pair(hA, hB,
                          src[p * EP6:(p + 1) * EP6],
                          dst[p * EP6:(p + 1) * EP6])
        gparts.append(run_k7(gsum_p).reshape(EP6))

    return jnp.concatenate(gparts).reshape(E, 1)
